# Initial kernel scaffold; baseline (speedup 1.0000x reference)
import jax
import jax.numpy as jnp
from jax.experimental import pallas as pl


def kernel(x, table):
    raise NotImplementedError("write your pallas kernel here")


# speedup vs baseline: 2.9661x; 2.9661x over previous
import functools

import jax
import jax.numpy as jnp
from jax import lax
from jax.experimental import pallas as pl
from jax.experimental.pallas import tpu as pltpu
from jax.experimental.pallas import tpu_sc as plsc

_info = plsc.get_sparse_core_info()
_NC, _NS, _L = _info.num_cores, _info.num_subcores, _info.num_lanes
_NW = _NC * _NS


def _make_gather(B, V, E, rows_per_worker, n_groups, group):
    mesh = plsc.VectorSubcoreMesh(core_axis_name="c", subcore_axis_name="s")

    @functools.partial(
        pl.kernel,
        mesh=mesh,
        out_type=jax.ShapeDtypeStruct((B, E), jnp.float32),
        scratch_types=[
            pltpu.VMEM((n_groups, group), jnp.int32),
            pltpu.VMEM((group, E), jnp.float32),
            pltpu.SemaphoreType.DMA,
        ],
    )
    def k(idx_hbm, table_hbm, out_hbm, idx_v, rows_v, sem):
        wid = lax.axis_index("s") * _NC + lax.axis_index("c")
        base = wid * rows_per_worker
        pltpu.sync_copy(idx_hbm.at[wid], idx_v)

        def body(j, _):
            pltpu.async_copy(table_hbm.at[idx_v.at[j]], rows_v, sem).wait()
            pltpu.sync_copy(rows_v, out_hbm.at[pl.ds(base + j * group, group)])
            return 0

        lax.fori_loop(0, n_groups, body, 0)

    return k


def kernel(x, table):
    Bo, S = x.shape
    V, E = table.shape
    B = Bo * S
    rows_per_worker = B // _NW
    group = 128
    n_groups = rows_per_worker // group
    idx = x.reshape(_NW, n_groups, group).astype(jnp.int32)
    out = _make_gather(B, V, E, rows_per_worker, n_groups, group)(idx, table)
    return out.reshape(Bo, S, E)


# speedup vs baseline: 3.3180x; 1.1186x over previous
import functools

import jax
import jax.numpy as jnp
from jax import lax
from jax.experimental import pallas as pl
from jax.experimental.pallas import tpu as pltpu
from jax.experimental.pallas import tpu_sc as plsc

_info = plsc.get_sparse_core_info()
_NC, _NS, _L = _info.num_cores, _info.num_subcores, _info.num_lanes
_NW = _NC * _NS

_CH = 128


def _make_gather(B, V, E, rows_per_worker, n_chunks):
    mesh = plsc.VectorSubcoreMesh(core_axis_name="c", subcore_axis_name="s")

    @functools.partial(
        pl.kernel,
        mesh=mesh,
        out_type=jax.ShapeDtypeStruct((B, E), jnp.float32),
        scratch_types=[
            pltpu.VMEM((n_chunks, _CH), jnp.int32),
            pltpu.VMEM((_CH, E), jnp.float32),
            pltpu.VMEM((_CH, E), jnp.float32),
            pltpu.SemaphoreType.DMA,
            pltpu.SemaphoreType.DMA,
            pltpu.SemaphoreType.DMA,
            pltpu.SemaphoreType.DMA,
        ],
    )
    def k(idx_hbm, table_hbm, out_hbm, idx_v, rows0, rows1, g0, g1, w0, w1):
        wid = lax.axis_index("s") * _NC + lax.axis_index("c")
        base = wid * rows_per_worker
        rows = (rows0, rows1)
        gsem = (g0, g1)
        wsem = (w0, w1)

        pltpu.sync_copy(idx_hbm.at[wid], idx_v)

        def gather(c, b):
            return pltpu.make_async_copy(
                table_hbm.at[idx_v.at[c]], rows[b], gsem[b]
            )

        def write(c, b):
            return pltpu.make_async_copy(
                rows[b], out_hbm.at[pl.ds(base + c * _CH, _CH)], wsem[b]
            )

        gather(0, 0).start()

        gather(1, 1).start()
        gather(0, 0).wait()
        write(0, 0).start()

        write(0, 0).wait()
        gather(2, 0).start()
        gather(1, 1).wait()
        write(1, 1).start()

        def body(i, _):
            g = i * 2
            for b in range(2):
                c = g + b
                write(c - 1, 1 - b).wait()
                gather(c + 1, 1 - b).start()
                gather(c, b).wait()
                write(c, b).start()
            return 0

        lax.fori_loop(1, n_chunks // 2 - 1, body, 0)

        cm2, cm1 = n_chunks - 2, n_chunks - 1
        write(cm2 - 1, 1).wait()
        gather(cm1, 1).start()
        gather(cm2, 0).wait()
        write(cm2, 0).start()

        write(cm2, 0).wait()
        gather(cm1, 1).wait()
        write(cm1, 1).start()
        write(cm1, 1).wait()

    return k


def kernel(x, table):
    Bo, S = x.shape
    V, E = table.shape
    B = Bo * S
    rows_per_worker = B // _NW
    n_chunks = rows_per_worker // _CH
    idx = x.reshape(_NW, n_chunks, _CH).astype(jnp.int32)
    out = _make_gather(B, V, E, rows_per_worker, n_chunks)(idx, table)
    return out.reshape(Bo, S, E)


# speedup vs baseline: 3.3635x; 1.0137x over previous
import functools

import jax
import jax.numpy as jnp
from jax import lax
from jax.experimental import pallas as pl
from jax.experimental.pallas import tpu as pltpu
from jax.experimental.pallas import tpu_sc as plsc

_info = plsc.get_sparse_core_info()
_NC, _NS, _L = _info.num_cores, _info.num_subcores, _info.num_lanes
_NW = _NC * _NS

_CH = 128
_NBUF = 6
_K = 3


def _make_gather(B, V, E, rows_per_worker, n_chunks):
    mesh = plsc.VectorSubcoreMesh(core_axis_name="c", subcore_axis_name="s")
    steady = ((n_chunks - _K - (_NBUF - _K)) // _NBUF) * _NBUF
    assert steady > 0

    @functools.partial(
        pl.kernel,
        mesh=mesh,
        out_type=jax.ShapeDtypeStruct((B, E), jnp.float32),
        scratch_types=[
            pltpu.VMEM((n_chunks, _CH), jnp.int32),
        ]
        + [pltpu.VMEM((_CH, E), jnp.float32) for _ in range(_NBUF)]
        + [pltpu.SemaphoreType.DMA for _ in range(2 * _NBUF)],
    )
    def k(idx_hbm, table_hbm, out_hbm, idx_v, *bufs):
        rows = bufs[:_NBUF]
        gsem = bufs[_NBUF : 2 * _NBUF]
        wsem = bufs[2 * _NBUF :]
        wid = lax.axis_index("s") * _NC + lax.axis_index("c")
        base = wid * rows_per_worker

        pltpu.sync_copy(idx_hbm.at[wid], idx_v)

        def gather(c, b):
            return pltpu.make_async_copy(
                table_hbm.at[idx_v.at[c]], rows[b], gsem[b]
            )

        def write(c, b):
            return pltpu.make_async_copy(
                rows[b], out_hbm.at[pl.ds(base + c * _CH, _CH)], wsem[b]
            )

        for j in range(_K):
            gather(j, j % _NBUF).start()

        for c in range(_K):
            gather(c + _K, (c + _K) % _NBUF).start()
            gather(c, c % _NBUF).wait()
            write(c, c % _NBUF).start()

        def body(i, _):
            g = i * _NBUF + _K
            for bb in range(_NBUF):
                c = g + bb
                b = (_K + bb) % _NBUF
                bn = (b + _K) % _NBUF
                write(c - (_NBUF - _K), bn).wait()
                gather(c + _K, bn).start()
                gather(c, b).wait()
                write(c, b).start()
            return 0

        lax.fori_loop(0, steady // _NBUF, body, 0)

        for c in range(_K + steady, n_chunks):
            b = c % _NBUF
            if c + _K < n_chunks:
                bn = (c + _K) % _NBUF
                write(c + _K - _NBUF, bn).wait()
                gather(c + _K, bn).start()
            gather(c, b).wait()
            write(c, b).start()

        hi = _K + steady
        lo = hi - (_NBUF - _K)
        waited = lo + max(0, n_chunks - _K - hi)
        for c in range(waited, n_chunks):
            write(c, c % _NBUF).wait()

    return k


def kernel(x, table):
    Bo, S = x.shape
    V, E = table.shape
    B = Bo * S
    rows_per_worker = B // _NW
    n_chunks = rows_per_worker // _CH
    idx = x.reshape(_NW, n_chunks, _CH).astype(jnp.int32)
    out = _make_gather(B, V, E, rows_per_worker, n_chunks)(idx, table)
    return out.reshape(Bo, S, E)
